# baseline (speedup 1.0000x reference)
import functools

import jax
import jax.numpy as jnp
from jax import lax
from jax.experimental import pallas as pl
from jax.experimental.pallas import tpu as pltpu
from jax.experimental.pallas import tpu_sc as plsc

_B, _L, _E = 1024, 200, 64
_BU = 128
_JU = _B // _BU
_NU = _L * _JU


@functools.cache
def _make_sc_gather_t(lh):
    info = plsc.get_sparse_core_info()
    nc, ns = info.num_cores, info.num_subcores
    nw = nc * ns
    upw = lh * _JU // nw
    nbuf = 3
    mesh = plsc.VectorSubcoreMesh(core_axis_name="c", subcore_axis_name="s",
                                  num_cores=nc)

    @functools.partial(
        pl.kernel,
        mesh=mesh,
        compiler_params=pltpu.CompilerParams(needs_layout_passes=False,
                                             use_tc_tiling_on_sc=False),
        out_type=jax.ShapeDtypeStruct((lh, _E // 8, _JU, 8, _BU), jnp.float32),
        scratch_types=[
            pltpu.VMEM((upw * _BU,), jnp.int32),
            pltpu.VMEM((_BU, _E), jnp.float32),
            pltpu.VMEM((_BU, _E), jnp.float32),
            pltpu.VMEM((_BU, _E), jnp.float32),
            pltpu.VMEM((_BU * (_E + 1),), jnp.float32),
            pltpu.VMEM((_E // 8, 8, _BU), jnp.float32),
            pltpu.VMEM((_E // 8, 8, _BU), jnp.float32),
            pltpu.VMEM((_E // 8, 8, _BU), jnp.float32),
            pltpu.SemaphoreType.DMA,
            pltpu.SemaphoreType.DMA,
            pltpu.SemaphoreType.DMA,
            pltpu.SemaphoreType.DMA,
            pltpu.SemaphoreType.DMA,
            pltpu.SemaphoreType.DMA,
            pltpu.SemaphoreType.DMA,
        ],
    )
    def sc_gather_t(ids_t_hbm, tok_hbm, out_hbm,
                    ids_all, rows0, rows1, rows2, stage65, t0, t1, t2,
                    sem_i, sg0, sg1, sg2, so0, so1, so2):
        rows_b = (rows0, rows1, rows2)
        tout_b = (t0, t1, t2)
        sem_g = (sg0, sg1, sg2)
        sem_o = (so0, so1, so2)

        wid = lax.axis_index("s") * nc + lax.axis_index("c")
        ubase = wid * upw

        for u in range(upw):
            gu = ubase + u
            pltpu.async_copy(
                ids_t_hbm.at[gu // _JU, pl.ds((gu % _JU) * _BU, _BU)],
                ids_all.at[pl.ds(u * _BU, _BU)], sem_i)
        for u in range(upw):
            pltpu.make_async_copy(ids_t_hbm.at[0, pl.ds(0, _BU)],
                                  ids_all.at[pl.ds(u * _BU, _BU)], sem_i).wait()

        def enq_gather(k, p):
            pltpu.async_copy(tok_hbm.at[ids_all.at[pl.ds(k * _BU, _BU)]],
                             rows_b[p], sem_g[p])

        def wait_gather(p):
            pltpu.make_async_copy(tok_hbm.at[ids_all.at[pl.ds(0, _BU)]],
                                  rows_b[p], sem_g[p]).wait()

        def enq_out(k, p):
            gu = ubase + k
            pltpu.async_copy(
                tout_b[p],
                out_hbm.at[gu // _JU, :, gu % _JU, :, :],
                sem_o[p])

        def wait_out(p):
            pltpu.make_async_copy(tout_b[p],
                                  out_hbm.at[0, :, 0, :, :],
                                  sem_o[p]).wait()

        lane = lax.iota(jnp.int32, 16)

        def transpose_unit(p):
            rows, tout = rows_b[p], tout_b[p]

            def repack(r, carry):
                for j in range(_E // 16):
                    stage65[pl.ds(r * (_E + 1) + 16 * j, 16)] = (
                        rows[r, pl.ds(16 * j, 16)])
                return carry

            lax.fori_loop(0, _BU, repack, 0, unroll=False)

            def tgroup(g, carry):
                idx0 = (g * 16 + lane) * (_E + 1)
                for e in range(_E):
                    v = plsc.load_gather(stage65, [idx0 + e])
                    tout[e // 8, e % 8, pl.ds(g * 16, 16)] = v
                return carry

            lax.fori_loop(0, _BU // 16, tgroup, 0, unroll=False)

        enq_gather(0, 0)
        enq_gather(1, 1)

        def unit_body(k, p, prefetch):
            wait_gather(p)
            if prefetch:
                @pl.when((k >= 1) & (k + 2 < upw))
                def _():
                    wait_out((p + 2) % nbuf)
                    enq_gather(k + 2, (p + 2) % nbuf)

                @pl.when((k < 1) & (k + 2 < upw))
                def _():
                    enq_gather(k + 2, (p + 2) % nbuf)
            transpose_unit(p)
            enq_out(k, p)

        def triple(i, carry):
            for c in range(3):
                unit_body(3 * i + c, c, True)
            return carry

        lax.fori_loop(0, upw // nbuf, triple, 0, unroll=False)
        for k in range(3 * (upw // nbuf), upw):
            unit_body(k, k % nbuf, False)
        wait_out((upw - 3) % nbuf)
        wait_out((upw - 2) % nbuf)
        wait_out((upw - 1) % nbuf)

    return sc_gather_t


def kernel(input_ids, segment_ids, token_table, pos_table, seg_table):
    lh = _L // 2
    ids_t = input_ids.T
    f = _make_sc_gather_t(lh)
    halves = []
    for h in range(2):
        tok5 = f(ids_t[h * lh:(h + 1) * lh], token_table)
        tok = jnp.transpose(tok5, (2, 4, 0, 1, 3)).reshape(_B, lh, _E)
        seg_ids = segment_ids[:, h * lh:(h + 1) * lh, None]
        seg = jnp.where(seg_ids == 0, seg_table[0], seg_table[1])
        halves.append(tok + pos_table[None, h * lh:(h + 1) * lh, :] + seg)
    return jnp.concatenate(halves, axis=1)


# speedup vs baseline: 1.0925x; 1.0925x over previous
import functools

import jax
import jax.numpy as jnp
from jax import lax
from jax.experimental import pallas as pl
from jax.experimental.pallas import tpu as pltpu
from jax.experimental.pallas import tpu_sc as plsc

_B, _L, _E = 1024, 200, 64
_BPC = 2
_CR = _BPC * _L


@functools.cache
def _make_sc_gather():
    info = plsc.get_sparse_core_info()
    nc, ns = info.num_cores, info.num_subcores
    nw = nc * ns
    bpw = _B // nw
    rpw = bpw * _L
    nch = bpw // _BPC
    nbuf = 3
    mesh = plsc.VectorSubcoreMesh(core_axis_name="c", subcore_axis_name="s",
                                  num_cores=nc)

    @functools.partial(
        pl.kernel,
        mesh=mesh,
        compiler_params=pltpu.CompilerParams(needs_layout_passes=False,
                                             use_tc_tiling_on_sc=False),
        out_type=jax.ShapeDtypeStruct((_B, _L, _E), jnp.float32),
        scratch_types=[
            pltpu.VMEM((rpw,), jnp.int32),
            pltpu.VMEM((_CR, _E), jnp.float32),
            pltpu.VMEM((_CR, _E), jnp.float32),
            pltpu.VMEM((_CR, _E), jnp.float32),
            pltpu.SemaphoreType.DMA,
            pltpu.SemaphoreType.DMA,
            pltpu.SemaphoreType.DMA,
            pltpu.SemaphoreType.DMA,
            pltpu.SemaphoreType.DMA,
            pltpu.SemaphoreType.DMA,
            pltpu.SemaphoreType.DMA,
        ],
    )
    def sc_gather(ids_hbm, tok_hbm, out_hbm,
                  ids_all, rows0, rows1, rows2,
                  sem_i, sg0, sg1, sg2, so0, so1, so2):
        rows_b = (rows0, rows1, rows2)
        sem_g = (sg0, sg1, sg2)
        sem_o = (so0, so1, so2)

        wid = lax.axis_index("s") * nc + lax.axis_index("c")
        bbase = wid * bpw

        for b in range(bpw):
            pltpu.async_copy(ids_hbm.at[bbase + b],
                             ids_all.at[pl.ds(b * _L, _L)], sem_i)
        for b in range(bpw):
            pltpu.make_async_copy(ids_hbm.at[0],
                                  ids_all.at[pl.ds(b * _L, _L)], sem_i).wait()

        def enq_gather(k, p):
            pltpu.async_copy(tok_hbm.at[ids_all.at[pl.ds(k * _CR, _CR)]],
                             rows_b[p], sem_g[p])

        def wait_gather(p):
            pltpu.make_async_copy(tok_hbm.at[ids_all.at[pl.ds(0, _CR)]],
                                  rows_b[p], sem_g[p]).wait()

        def enq_out(k, p):
            b = bbase + _BPC * k
            for i in range(_BPC):
                pltpu.async_copy(rows_b[p].at[pl.ds(i * _L, _L)],
                                 out_hbm.at[b + i], sem_o[p])

        def wait_out(p):
            for i in range(_BPC):
                pltpu.make_async_copy(rows_b[p].at[pl.ds(i * _L, _L)],
                                      out_hbm.at[0], sem_o[p]).wait()

        enq_gather(0, 0)
        enq_gather(1, 1)

        def chunk_body(k, p, prefetch):
            wait_gather(p)
            if prefetch:
                @pl.when((k >= 1) & (k + 2 < nch))
                def _():
                    wait_out((p + 2) % nbuf)
                    enq_gather(k + 2, (p + 2) % nbuf)

                @pl.when((k < 1) & (k + 2 < nch))
                def _():
                    enq_gather(k + 2, (p + 2) % nbuf)
            enq_out(k, p)

        def triple(i, carry):
            for c in range(3):
                chunk_body(3 * i + c, c, True)
            return carry

        lax.fori_loop(0, nch // nbuf, triple, 0, unroll=False)
        chunk_body(nch - 1, (nch - 1) % nbuf, False)
        wait_out((nch - 3) % nbuf)
        wait_out((nch - 2) % nbuf)
        wait_out((nch - 1) % nbuf)

    return sc_gather


def kernel(input_ids, segment_ids, token_table, pos_table, seg_table):
    tok = _make_sc_gather()(input_ids, token_table)
    seg = jnp.where(segment_ids[:, :, None] == 0, seg_table[0], seg_table[1])
    return tok + pos_table[None, :_L, :] + seg
